# baseline (speedup 1.0000x reference)
import functools

import jax
import jax.numpy as jnp
from jax import lax
from jax.experimental import pallas as pl
from jax.experimental.pallas import tpu as pltpu
from jax.experimental.pallas import tpu_sc as plsc

_DIM = 2048
_NE = 64
_NTOK = 16384
_NW = 32
_TPW = _NTOK // _NW
_HALF = _TPW // 2
_LANES = 16
_TC_TILE = 1024
_SLABS_PER_STEP = _TC_TILE // _TPW


def _logits_body(x_ref, w_ref, b_ref, out_ref):
    x = x_ref[...]
    w = w_ref[...]
    b = b_ref[...]
    logits = lax.dot_general(w, x, (((1,), (1,)), ((), ())),
                             preferred_element_type=jnp.float32) + b
    for s in range(_SLABS_PER_STEP):
        out_ref[s] = logits[:, s * _TPW:(s + 1) * _TPW]


def _tc_logits(x2, W, b2):
    return pl.pallas_call(
        _logits_body,
        grid=(_NTOK // _TC_TILE,),
        in_specs=[
            pl.BlockSpec((_TC_TILE, _DIM), lambda i: (i, 0)),
            pl.BlockSpec((_NE, _DIM), lambda i: (0, 0)),
            pl.BlockSpec((_NE, 1), lambda i: (0, 0)),
        ],
        out_specs=pl.BlockSpec((_SLABS_PER_STEP, _NE, _TPW), lambda i: (i, 0, 0)),
        out_shape=jax.ShapeDtypeStruct((_NW, _NE, _TPW), jnp.float32),
    )(x2, W, b2)


@functools.partial(
    pl.kernel,
    mesh=plsc.VectorSubcoreMesh(core_axis_name="c", subcore_axis_name="s"),
    out_type=jax.ShapeDtypeStruct((_NW, 4, _TPW), jnp.float32),
    scratch_types=[
        pltpu.VMEM((_NE, _TPW), jnp.float32),
        pltpu.VMEM((4, _TPW), jnp.float32),
        pltpu.SemaphoreType.DMA,
        pltpu.SemaphoreType.DMA,
    ],
)
def _sc_route(logits_hbm, out_hbm, slab, ob, sem0, sem1):
    wid = lax.axis_index("s") * 2 + lax.axis_index("c")
    cp0 = pltpu.async_copy(
        logits_hbm.at[wid, :, pl.ds(0, _HALF)], slab.at[:, pl.ds(0, _HALF)], sem0)
    cp1 = pltpu.async_copy(
        logits_hbm.at[wid, :, pl.ds(_HALF, _HALF)],
        slab.at[:, pl.ds(_HALF, _HALF)], sem1)

    def chunk(c, carry):
        off = pl.multiple_of(c * _LANES, _LANES)
        m1 = jnp.full((_LANES,), -jnp.inf, jnp.float32)
        m2 = jnp.full((_LANES,), -jnp.inf, jnp.float32)
        i1 = jnp.zeros((_LANES,), jnp.float32)
        i2 = jnp.zeros((_LANES,), jnp.float32)
        for e in range(_NE):
            v = slab[e, pl.ds(off, _LANES)]
            gt1 = v > m1
            gt2 = v > m2
            m2 = jnp.where(gt1, m1, jnp.where(gt2, v, m2))
            i2 = jnp.where(gt1, i1, jnp.where(gt2, float(e), i2))
            m1 = jnp.where(gt1, v, m1)
            i1 = jnp.where(gt1, float(e), i1)
        s1 = 1.0 / (1.0 + jnp.exp(m2 - m1))
        ob[0, pl.ds(off, _LANES)] = i1
        ob[1, pl.ds(off, _LANES)] = i2
        ob[2, pl.ds(off, _LANES)] = s1
        ob[3, pl.ds(off, _LANES)] = 1.0 - s1
        return carry

    cp0.wait()
    lax.fori_loop(0, _HALF // _LANES, chunk, 0)
    cp1.wait()
    lax.fori_loop(_HALF // _LANES, _TPW // _LANES, chunk, 0)
    pltpu.sync_copy(ob, out_hbm.at[wid])


def kernel(x, W, b):
    bsz, seq, dim = x.shape
    n_tok = bsz * seq
    x2 = x.reshape(n_tok, dim)
    b2 = b.reshape(_NE, 1)
    logits_t = _tc_logits(x2, W, b2)
    routed = _sc_route(logits_t)
    flat = routed.transpose(0, 2, 1).reshape(n_tok, 4)
    idx = flat[:, :2].astype(jnp.int32).reshape(bsz, seq, 2)
    scr = flat[:, 2:].reshape(bsz, seq, 2)
    return (idx, scr)


# speedup vs baseline: 1.0189x; 1.0189x over previous
import functools

import jax
import jax.numpy as jnp
from jax import lax
from jax.experimental import pallas as pl
from jax.experimental.pallas import tpu as pltpu
from jax.experimental.pallas import tpu_sc as plsc

_DIM = 2048
_NE = 64
_NTOK = 16384
_NW = 32
_TPW = _NTOK // _NW
_HALF = _TPW // 2
_LANES = 16
_TC_TILE = 1024
_SLABS_PER_STEP = _TC_TILE // _TPW


def _logits_body(x_ref, w_ref, b_ref, out_ref):
    x = x_ref[...]
    w = w_ref[...]
    b = b_ref[...]
    logits = lax.dot_general(w, x, (((1,), (1,)), ((), ())),
                             preferred_element_type=jnp.float32) + b
    for s in range(_SLABS_PER_STEP):
        out_ref[s] = logits[:, s * _TPW:(s + 1) * _TPW]


def _tc_logits(x2, W, b2):
    return pl.pallas_call(
        _logits_body,
        grid=(_NTOK // _TC_TILE,),
        in_specs=[
            pl.BlockSpec((_TC_TILE, _DIM), lambda i: (i, 0)),
            pl.BlockSpec((_NE, _DIM), lambda i: (0, 0)),
            pl.BlockSpec((_NE, 1), lambda i: (0, 0)),
        ],
        out_specs=pl.BlockSpec((_SLABS_PER_STEP, _NE, _TPW), lambda i: (i, 0, 0)),
        out_shape=jax.ShapeDtypeStruct((_NW, _NE, _TPW), jnp.float32),
    )(x2, W, b2)


@functools.partial(
    pl.kernel,
    mesh=plsc.VectorSubcoreMesh(core_axis_name="c", subcore_axis_name="s"),
    out_type=jax.ShapeDtypeStruct((_NW, 4, _TPW), jnp.float32),
    scratch_types=[
        pltpu.VMEM((_NE, _TPW), jnp.float32),
        pltpu.VMEM((4, _TPW), jnp.float32),
        pltpu.SemaphoreType.DMA,
        pltpu.SemaphoreType.DMA,
    ],
)
def _sc_route(logits_hbm, out_hbm, slab, ob, sem0, sem1):
    wid = lax.axis_index("s") * 2 + lax.axis_index("c")
    pltpu.sync_copy(logits_hbm.at[wid], slab)

    def chunk(c, carry):
        off = pl.multiple_of(c * _LANES, _LANES)
        m1 = jnp.full((_LANES,), -jnp.inf, jnp.float32)
        m2 = jnp.full((_LANES,), -jnp.inf, jnp.float32)
        i1 = jnp.zeros((_LANES,), jnp.float32)
        i2 = jnp.zeros((_LANES,), jnp.float32)
        for e in range(_NE):
            v = slab[e, pl.ds(off, _LANES)]
            gt1 = v > m1
            gt2 = v > m2
            m2 = jnp.where(gt1, m1, jnp.where(gt2, v, m2))
            i2 = jnp.where(gt1, i1, jnp.where(gt2, float(e), i2))
            m1 = jnp.where(gt1, v, m1)
            i1 = jnp.where(gt1, float(e), i1)
        s1 = 1.0 / (1.0 + jnp.exp(m2 - m1))
        ob[0, pl.ds(off, _LANES)] = i1
        ob[1, pl.ds(off, _LANES)] = i2
        ob[2, pl.ds(off, _LANES)] = s1
        ob[3, pl.ds(off, _LANES)] = 1.0 - s1
        return carry

    lax.fori_loop(0, _TPW // _LANES, chunk, 0)
    pltpu.sync_copy(ob, out_hbm.at[wid])


def kernel(x, W, b):
    bsz, seq, dim = x.shape
    n_tok = bsz * seq
    x2 = x.reshape(n_tok, dim)
    b2 = b.reshape(_NE, 1)
    logits_t = _tc_logits(x2, W, b2)
    routed = _sc_route(logits_t)
    flat = routed.transpose(0, 2, 1).reshape(n_tok, 4)
    idx = flat[:, :2].astype(jnp.int32).reshape(bsz, seq, 2)
    scr = flat[:, 2:].reshape(bsz, seq, 2)
    return (idx, scr)


# speedup vs baseline: 1.0474x; 1.0279x over previous
import functools

import jax
import jax.numpy as jnp
from jax import lax
from jax.experimental import pallas as pl
from jax.experimental.pallas import tpu as pltpu
from jax.experimental.pallas import tpu_sc as plsc

_DIM = 2048
_NE = 64
_NTOK = 16384
_NW = 32
_TPW = _NTOK // _NW
_LANES = 16
_TC_TILE = 1024
_SLABS_PER_STEP = _TC_TILE // _TPW


def _logits_body(x_ref, w_ref, b_ref, out_ref):
    x = x_ref[...]
    w = w_ref[...]
    b = b_ref[...]
    logits = lax.dot_general(w, x, (((1,), (1,)), ((), ())),
                             preferred_element_type=jnp.float32) + b
    for s in range(_SLABS_PER_STEP):
        out_ref[s] = logits[:, s * _TPW:(s + 1) * _TPW]


def _tc_logits(x2, W, b2):
    return pl.pallas_call(
        _logits_body,
        grid=(_NTOK // _TC_TILE,),
        in_specs=[
            pl.BlockSpec((_TC_TILE, _DIM), lambda i: (i, 0)),
            pl.BlockSpec((_NE, _DIM), lambda i: (0, 0)),
            pl.BlockSpec((_NE, 1), lambda i: (0, 0)),
        ],
        out_specs=pl.BlockSpec((_SLABS_PER_STEP, _NE, _TPW), lambda i: (i, 0, 0)),
        out_shape=jax.ShapeDtypeStruct((_NW, _NE, _TPW), jnp.float32),
    )(x2, W, b2)


def _top2_update(v, e, m1, m2, i1, i2):
    gt1 = v > m1
    gt2 = v > m2
    m2 = jnp.where(gt1, m1, jnp.where(gt2, v, m2))
    i2 = jnp.where(gt1, i1, jnp.where(gt2, e, i2))
    m1 = jnp.where(gt1, v, m1)
    i1 = jnp.where(gt1, e, i1)
    return m1, m2, i1, i2


@functools.partial(
    pl.kernel,
    mesh=plsc.VectorSubcoreMesh(core_axis_name="c", subcore_axis_name="s"),
    out_type=[
        jax.ShapeDtypeStruct((2, _NTOK), jnp.int32),
        jax.ShapeDtypeStruct((2, _NTOK), jnp.float32),
    ],
    scratch_types=[
        pltpu.VMEM((_NE, _TPW), jnp.float32),
        pltpu.VMEM((_TPW,), jnp.int32),
        pltpu.VMEM((_TPW,), jnp.int32),
        pltpu.VMEM((_TPW,), jnp.float32),
        pltpu.VMEM((_TPW,), jnp.float32),
    ],
)
def _sc_route(logits_hbm, idx_hbm, scr_hbm, slab, i1b, i2b, s1b, s2b):
    wid = lax.axis_index("s") * 2 + lax.axis_index("c")
    pltpu.sync_copy(logits_hbm.at[wid], slab)

    neg_inf = jnp.full((_LANES,), -jnp.inf, jnp.float32)
    zero_i = jnp.zeros((_LANES,), jnp.int32)

    def chunk(c, carry):
        off = pl.multiple_of(c * _LANES, _LANES)
        a_m1, a_m2, a_i1, a_i2 = neg_inf, neg_inf, zero_i, zero_i
        b_m1, b_m2, b_i1, b_i2 = neg_inf, neg_inf, zero_i, zero_i
        half = _NE // 2
        for e in range(half):
            va = slab[e, pl.ds(off, _LANES)]
            vb = slab[e + half, pl.ds(off, _LANES)]
            a_m1, a_m2, a_i1, a_i2 = _top2_update(va, e, a_m1, a_m2, a_i1, a_i2)
            b_m1, b_m2, b_i1, b_i2 = _top2_update(vb, e + half, b_m1, b_m2,
                                                  b_i1, b_i2)
        m1, m2, i1, i2 = a_m1, a_m2, a_i1, a_i2
        m1, m2, i1, i2 = _top2_update(b_m1, b_i1, m1, m2, i1, i2)
        m1, m2, i1, i2 = _top2_update(b_m2, b_i2, m1, m2, i1, i2)
        s1 = 1.0 / (1.0 + jnp.exp(m2 - m1))
        i1b[pl.ds(off, _LANES)] = i1
        i2b[pl.ds(off, _LANES)] = i2
        s1b[pl.ds(off, _LANES)] = s1
        s2b[pl.ds(off, _LANES)] = 1.0 - s1
        return carry

    lax.fori_loop(0, _TPW // _LANES, chunk, 0)
    base = wid * _TPW
    pltpu.sync_copy(i1b, idx_hbm.at[0, pl.ds(base, _TPW)])
    pltpu.sync_copy(i2b, idx_hbm.at[1, pl.ds(base, _TPW)])
    pltpu.sync_copy(s1b, scr_hbm.at[0, pl.ds(base, _TPW)])
    pltpu.sync_copy(s2b, scr_hbm.at[1, pl.ds(base, _TPW)])


def kernel(x, W, b):
    bsz, seq, dim = x.shape
    n_tok = bsz * seq
    x2 = x.reshape(n_tok, dim)
    b2 = b.reshape(_NE, 1)
    logits_t = _tc_logits(x2, W, b2)
    idx_t, scr_t = _sc_route(logits_t)
    idx = idx_t.T.reshape(bsz, seq, 2)
    scr = scr_t.T.reshape(bsz, seq, 2)
    return (idx, scr)


# speedup vs baseline: 1.0498x; 1.0023x over previous
import functools

import jax
import jax.numpy as jnp
from jax import lax
from jax.experimental import pallas as pl
from jax.experimental.pallas import tpu as pltpu
from jax.experimental.pallas import tpu_sc as plsc

_DIM = 2048
_NE = 64
_NTOK = 16384
_NW = 32
_TPW = _NTOK // _NW
_LANES = 16
_TC_TILE = 1024
_SLABS_PER_STEP = _TC_TILE // _TPW


def _logits_body(x_ref, w_ref, b_ref, out_ref):
    x = x_ref[...]
    w = w_ref[...]
    b = b_ref[...]
    logits = lax.dot_general(w, x, (((1,), (1,)), ((), ())),
                             preferred_element_type=jnp.float32) + b
    for s in range(_SLABS_PER_STEP):
        out_ref[s] = logits[:, s * _TPW:(s + 1) * _TPW]


def _tc_logits(x2, W, b2):
    return pl.pallas_call(
        _logits_body,
        grid=(_NTOK // _TC_TILE,),
        in_specs=[
            pl.BlockSpec((_TC_TILE, _DIM), lambda i: (i, 0)),
            pl.BlockSpec((_NE, _DIM), lambda i: (0, 0)),
            pl.BlockSpec((_NE, 1), lambda i: (0, 0)),
        ],
        out_specs=pl.BlockSpec((_SLABS_PER_STEP, _NE, _TPW), lambda i: (i, 0, 0)),
        out_shape=jax.ShapeDtypeStruct((_NW, _NE, _TPW), jnp.float32),
    )(x2, W, b2)


def _top2_update(v, e, m1, m2, i1, i2):
    gt1 = v > m1
    gt2 = v > m2
    m2 = jnp.where(gt1, m1, jnp.where(gt2, v, m2))
    i2 = jnp.where(gt1, i1, jnp.where(gt2, e, i2))
    m1 = jnp.where(gt1, v, m1)
    i1 = jnp.where(gt1, e, i1)
    return m1, m2, i1, i2


@functools.partial(
    pl.kernel,
    mesh=plsc.VectorSubcoreMesh(core_axis_name="c", subcore_axis_name="s"),
    out_type=[
        jax.ShapeDtypeStruct((2, _NTOK), jnp.int32),
        jax.ShapeDtypeStruct((2, _NTOK), jnp.float32),
    ],
    scratch_types=[
        pltpu.VMEM((_NE, _TPW), jnp.float32),
        pltpu.VMEM((_TPW,), jnp.int32),
        pltpu.VMEM((_TPW,), jnp.int32),
        pltpu.VMEM((_TPW,), jnp.float32),
        pltpu.VMEM((_TPW,), jnp.float32),
        pltpu.SemaphoreType.DMA,
        pltpu.SemaphoreType.DMA,
        pltpu.SemaphoreType.DMA,
        pltpu.SemaphoreType.DMA,
    ],
)
def _sc_route(logits_hbm, idx_hbm, scr_hbm, slab, i1b, i2b, s1b, s2b,
              d0, d1, d2, d3):
    wid = lax.axis_index("s") * 2 + lax.axis_index("c")
    pltpu.sync_copy(logits_hbm.at[wid], slab)

    neg_inf = jnp.full((_LANES,), -jnp.inf, jnp.float32)
    zero_i = jnp.zeros((_LANES,), jnp.int32)

    def chunk(c, carry):
        off = pl.multiple_of(c * _LANES, _LANES)
        a_m1, a_m2, a_i1, a_i2 = neg_inf, neg_inf, zero_i, zero_i
        b_m1, b_m2, b_i1, b_i2 = neg_inf, neg_inf, zero_i, zero_i
        half = _NE // 2
        for e in range(half):
            va = slab[e, pl.ds(off, _LANES)]
            vb = slab[e + half, pl.ds(off, _LANES)]
            a_m1, a_m2, a_i1, a_i2 = _top2_update(va, e, a_m1, a_m2, a_i1, a_i2)
            b_m1, b_m2, b_i1, b_i2 = _top2_update(vb, e + half, b_m1, b_m2,
                                                  b_i1, b_i2)
        m1, m2, i1, i2 = a_m1, a_m2, a_i1, a_i2
        m1, m2, i1, i2 = _top2_update(b_m1, b_i1, m1, m2, i1, i2)
        m1, m2, i1, i2 = _top2_update(b_m2, b_i2, m1, m2, i1, i2)
        s1 = 1.0 / (1.0 + jnp.exp(m2 - m1))
        i1b[pl.ds(off, _LANES)] = i1
        i2b[pl.ds(off, _LANES)] = i2
        s1b[pl.ds(off, _LANES)] = s1
        s2b[pl.ds(off, _LANES)] = 1.0 - s1
        return carry

    lax.fori_loop(0, _TPW // _LANES, chunk, 0)
    base = wid * _TPW
    c0 = pltpu.async_copy(i1b, idx_hbm.at[0, pl.ds(base, _TPW)], d0)
    c1 = pltpu.async_copy(i2b, idx_hbm.at[1, pl.ds(base, _TPW)], d1)
    c2 = pltpu.async_copy(s1b, scr_hbm.at[0, pl.ds(base, _TPW)], d2)
    c3 = pltpu.async_copy(s2b, scr_hbm.at[1, pl.ds(base, _TPW)], d3)
    c0.wait()
    c1.wait()
    c2.wait()
    c3.wait()


def kernel(x, W, b):
    bsz, seq, dim = x.shape
    n_tok = bsz * seq
    x2 = x.reshape(n_tok, dim)
    b2 = b.reshape(_NE, 1)
    logits_t = _tc_logits(x2, W, b2)
    idx_t, scr_t = _sc_route(logits_t)
    idx = idx_t.T.reshape(bsz, seq, 2)
    scr = scr_t.T.reshape(bsz, seq, 2)
    return (idx, scr)
